# baseline (speedup 1.0000x reference)
import jax
import jax.numpy as jnp
from jax import lax
from jax.experimental import pallas as pl
from jax.experimental.pallas import tpu as pltpu
from jax.experimental.pallas import tpu_sc as plsc

N = 16384
C = 256
BATCH = 16
GROUP = 8
NG = N // GROUP

NUM_CORES = 2
NUM_SUBCORES = 16
NW = NUM_CORES * NUM_SUBCORES
LANES = 16
CVEC = C // LANES

TC_STEPS = 16
GB = NG // TC_STEPS

NEG = float("-inf")


def _tc_bounds_body(ids_ref, out_ref):
    ids2 = ids_ref[...]
    vals = [jnp.int32(0)]
    for s in range(1, BATCH):
        vals.append(jnp.sum((ids2 < s).astype(jnp.int32)))
    row = jnp.stack(vals)
    out_ref[...] = jnp.broadcast_to(row[None, :], (8, BATCH))


def _tc_dense_body(p3_ref, ids8_ref, out_ref):
    i = pl.program_id(0)

    @pl.when(i == 0)
    def _():
        out_ref[...] = jnp.full((BATCH, C), NEG, jnp.float32)

    x = p3_ref[...]
    g = x[:, 0, :]
    for j in range(1, GROUP):
        g = jnp.maximum(g, x[:, j, :])
    idm = ids8_ref[:, 0]
    idM = ids8_ref[:, GROUP - 1]
    for s in range(BATCH):
        pure = (idm == s) & (idM == s)
        contr = jnp.max(jnp.where(pure[:, None], g, NEG), axis=0)
        out_ref[s, :] = jnp.maximum(out_ref[s, :], contr)


def _sc_boundary_body(patches_hbm, bounds_hbm, parts_hbm, bv_v, buf, acc1,
                      sem0):
    wid = lax.axis_index("s") * NUM_CORES + lax.axis_index("c")
    seg = lax.shift_right_logical(wid, 1)
    h = lax.bitwise_and(wid, 1)

    pltpu.sync_copy(bounds_hbm.at[0], bv_v)
    bv = bv_v[pl.ds(0, BATCH)]

    neg = jnp.full((LANES,), NEG, jnp.float32)
    for c in range(CVEC):
        acc1[pl.ds(c * LANES, LANES)] = neg

    def reduce_rows(lo, hi):
        init = tuple(acc1[pl.ds(c * LANES, LANES)] for c in range(CVEC))

        def fbody(r, carry):
            return tuple(
                jnp.maximum(carry[c], buf[r, pl.ds(c * LANES, LANES)])
                for c in range(CVEC))
        res = lax.fori_loop(lo, hi, fbody, init)
        for c in range(CVEC):
            acc1[pl.ds(c * LANES, LANES)] = res[c]

    for sc in range(BATCH):
        b0 = bv[sc]
        b1 = bv[sc + 1] if sc < BATCH - 1 else jnp.int32(N)
        gstart = lax.shift_right_logical(b0 + (GROUP - 1), 3)
        gend = lax.shift_right_logical(b1, 3)
        e = jnp.clip(GROUP * gstart, b0, b1)
        f = jnp.clip(GROUP * gend, e, b1)

        @pl.when((seg == sc) & (h == 0) & (e > b0))
        def _head(b0=b0, e=e):
            gh = lax.shift_right_logical(b0, 3)
            pltpu.async_copy(
                patches_hbm.at[pl.ds(GROUP * gh, GROUP)], buf, sem0).wait()
            reduce_rows(b0 - GROUP * gh, e - GROUP * gh)

        @pl.when((seg == sc) & (h == 1) & (b1 > f))
        def _tail(b1=b1, f=f):
            gt = lax.shift_right_logical(b1 - 1, 3)
            pltpu.async_copy(
                patches_hbm.at[pl.ds(GROUP * gt, GROUP)], buf, sem0).wait()
            reduce_rows(f - GROUP * gt, b1 - GROUP * gt)

    pltpu.sync_copy(acc1, parts_hbm.at[wid])


def _tc_final_body(tcp_ref, parts_ref, out_ref):
    p = parts_ref[...]
    q = jnp.maximum(p[:, 0, :], p[:, 1, :])
    out_ref[...] = jnp.maximum(tcp_ref[...], q)


_sc_boundary = pl.kernel(
    _sc_boundary_body,
    out_type=jax.ShapeDtypeStruct((NW, C), jnp.float32),
    mesh=plsc.VectorSubcoreMesh(core_axis_name="c", subcore_axis_name="s",
                                num_cores=NUM_CORES,
                                num_subcores=NUM_SUBCORES),
    scratch_types=[
        pltpu.VMEM((BATCH,), jnp.int32),
        pltpu.VMEM((GROUP, C), jnp.float32),
        pltpu.VMEM((C,), jnp.float32),
        pltpu.SemaphoreType.DMA,
    ],
)


def kernel(patches, patch_ids):
    ids2 = patch_ids.reshape(128, 128)
    bounds = pl.pallas_call(
        _tc_bounds_body,
        out_shape=jax.ShapeDtypeStruct((8, BATCH), jnp.int32),
    )(ids2)

    parts = _sc_boundary(patches, bounds)

    p3 = patches.reshape(NG, GROUP, C)
    ids8 = patch_ids.reshape(NG, GROUP)
    tcp = pl.pallas_call(
        _tc_dense_body,
        grid=(TC_STEPS,),
        in_specs=[
            pl.BlockSpec((GB, GROUP, C), lambda i: (i, 0, 0)),
            pl.BlockSpec((GB, GROUP), lambda i: (i, 0)),
        ],
        out_specs=pl.BlockSpec((BATCH, C), lambda i: (0, 0)),
        out_shape=jax.ShapeDtypeStruct((BATCH, C), jnp.float32),
    )(p3, ids8)

    parts3 = parts.reshape(BATCH, 2, C)
    out = pl.pallas_call(
        _tc_final_body,
        out_shape=jax.ShapeDtypeStruct((BATCH, C), jnp.float32),
    )(tcp, parts3)
    return out


# speedup vs baseline: 1.0992x; 1.0992x over previous
import jax
import jax.numpy as jnp
from jax import lax
from jax.experimental import pallas as pl
from jax.experimental.pallas import tpu as pltpu
from jax.experimental.pallas import tpu_sc as plsc

N = 16384
C = 256
BATCH = 16
GROUP = 8
NG = N // GROUP

NUM_CORES = 2
NUM_SUBCORES = 16
NW = NUM_CORES * NUM_SUBCORES
LANES = 16
CVEC = C // LANES

TC_STEPS = 16
GB = NG // TC_STEPS

NEG = float("-inf")


def _tc_bounds_body(ids_ref, out_ref):
    ids2 = ids_ref[...]
    vals = [jnp.int32(0)]
    for s in range(1, BATCH):
        vals.append(jnp.sum((ids2 < s).astype(jnp.int32)))
    row = jnp.stack(vals)
    out_ref[...] = jnp.broadcast_to(row[None, :], (8, BATCH))


def _tc_dense_body(p3_ref, ids8_ref, out_ref):
    i = pl.program_id(0)

    @pl.when(i == 0)
    def _():
        out_ref[...] = jnp.full((BATCH, C), NEG, jnp.float32)

    x = p3_ref[...]
    g = jnp.max(x, axis=1)
    idm = ids8_ref[:, 0]
    idM = ids8_ref[:, GROUP - 1]
    for s in range(BATCH):
        pure = (idm == s) & (idM == s)
        contr = jnp.max(jnp.where(pure[:, None], g, NEG), axis=0)
        out_ref[s, :] = jnp.maximum(out_ref[s, :], contr)


def _sc_boundary_body(patches_hbm, bounds_hbm, parts_hbm, bv_v, buf, acc1,
                      sem0):
    wid = lax.axis_index("s") * NUM_CORES + lax.axis_index("c")
    seg = lax.shift_right_logical(wid, 1)
    h = lax.bitwise_and(wid, 1)

    pltpu.sync_copy(bounds_hbm.at[0], bv_v)
    bv = bv_v[pl.ds(0, BATCH)]

    neg = jnp.full((LANES,), NEG, jnp.float32)
    for c in range(CVEC):
        acc1[pl.ds(c * LANES, LANES)] = neg

    def reduce_rows(lo, hi):
        init = tuple(acc1[pl.ds(c * LANES, LANES)] for c in range(CVEC))

        def fbody(r, carry):
            return tuple(
                jnp.maximum(carry[c], buf[r, pl.ds(c * LANES, LANES)])
                for c in range(CVEC))
        res = lax.fori_loop(lo, hi, fbody, init)
        for c in range(CVEC):
            acc1[pl.ds(c * LANES, LANES)] = res[c]

    for sc in range(BATCH):
        b0 = bv[sc]
        b1 = bv[sc + 1] if sc < BATCH - 1 else jnp.int32(N)
        gstart = lax.shift_right_logical(b0 + (GROUP - 1), 3)
        gend = lax.shift_right_logical(b1, 3)
        e = jnp.clip(GROUP * gstart, b0, b1)
        f = jnp.clip(GROUP * gend, e, b1)

        @pl.when((seg == sc) & (h == 0) & (e > b0))
        def _head(b0=b0, e=e):
            gh = lax.shift_right_logical(b0, 3)
            pltpu.async_copy(
                patches_hbm.at[pl.ds(GROUP * gh, GROUP)], buf, sem0).wait()
            reduce_rows(b0 - GROUP * gh, e - GROUP * gh)

        @pl.when((seg == sc) & (h == 1) & (b1 > f))
        def _tail(b1=b1, f=f):
            gt = lax.shift_right_logical(b1 - 1, 3)
            pltpu.async_copy(
                patches_hbm.at[pl.ds(GROUP * gt, GROUP)], buf, sem0).wait()
            reduce_rows(f - GROUP * gt, b1 - GROUP * gt)

    pltpu.sync_copy(acc1, parts_hbm.at[wid])


def _tc_final_body(tcp_ref, parts_ref, out_ref):
    p = parts_ref[...]
    q = jnp.maximum(p[:, 0, :], p[:, 1, :])
    out_ref[...] = jnp.maximum(tcp_ref[...], q)


_sc_boundary = pl.kernel(
    _sc_boundary_body,
    out_type=jax.ShapeDtypeStruct((NW, C), jnp.float32),
    mesh=plsc.VectorSubcoreMesh(core_axis_name="c", subcore_axis_name="s",
                                num_cores=NUM_CORES,
                                num_subcores=NUM_SUBCORES),
    scratch_types=[
        pltpu.VMEM((BATCH,), jnp.int32),
        pltpu.VMEM((GROUP, C), jnp.float32),
        pltpu.VMEM((C,), jnp.float32),
        pltpu.SemaphoreType.DMA,
    ],
)


def kernel(patches, patch_ids):
    ids2 = patch_ids.reshape(128, 128)
    bounds = pl.pallas_call(
        _tc_bounds_body,
        out_shape=jax.ShapeDtypeStruct((8, BATCH), jnp.int32),
    )(ids2)

    parts = _sc_boundary(patches, bounds)

    p3 = patches.reshape(NG, GROUP, C)
    ids8 = patch_ids.reshape(NG, GROUP)
    tcp = pl.pallas_call(
        _tc_dense_body,
        grid=(TC_STEPS,),
        in_specs=[
            pl.BlockSpec((GB, GROUP, C), lambda i: (i, 0, 0)),
            pl.BlockSpec((GB, GROUP), lambda i: (i, 0)),
        ],
        out_specs=pl.BlockSpec((BATCH, C), lambda i: (0, 0)),
        out_shape=jax.ShapeDtypeStruct((BATCH, C), jnp.float32),
    )(p3, ids8)

    parts3 = parts.reshape(BATCH, 2, C)
    out = pl.pallas_call(
        _tc_final_body,
        out_shape=jax.ShapeDtypeStruct((BATCH, C), jnp.float32),
    )(tcp, parts3)
    return out


# speedup vs baseline: 1.3829x; 1.2581x over previous
import jax
import jax.numpy as jnp
from jax import lax
from jax.experimental import pallas as pl
from jax.experimental.pallas import tpu as pltpu
from jax.experimental.pallas import tpu_sc as plsc

N = 16384
C = 256
BATCH = 16

NUM_CORES = 2
NUM_SUBCORES = 16
NW = NUM_CORES * NUM_SUBCORES
ROWS = N // NW
CHUNK = 128
NCHUNK = ROWS // CHUNK
LANES = 16
CVEC = C // LANES


def _sc_body(patches_hbm, ids_hbm, parts_hbm, ids_v, buf0, buf1, acc_v,
             sem0, sem1):
    wid = lax.axis_index("s") * NUM_CORES + lax.axis_index("c")
    base = wid * ROWS

    pltpu.sync_copy(ids_hbm.at[pl.ds(base, ROWS)], ids_v)
    pltpu.async_copy(patches_hbm.at[pl.ds(base, CHUNK)], buf0, sem0)
    pltpu.async_copy(patches_hbm.at[pl.ds(base + CHUNK, CHUNK)], buf1, sem1)

    neg = jnp.full((LANES,), -jnp.inf, dtype=jnp.float32)

    def ibody(s, _):
        def icb(c, _):
            acc_v[s, pl.ds(c * LANES, LANES)] = neg
            return 0
        return lax.fori_loop(0, CVEC, icb, 0)
    lax.fori_loop(0, BATCH, ibody, 0)

    def cbody(j, cnt):
        v = ids_v[pl.ds(j * LANES, LANES)]
        return tuple(cnt[s - 1] + jnp.minimum(jnp.maximum(s - v, 0), 1)
                     for s in range(1, BATCH))
    cnt = lax.fori_loop(
        0, ROWS // LANES, cbody,
        tuple(jnp.zeros((LANES,), jnp.int32) for _ in range(BATCH - 1)))
    iota = lax.iota(jnp.int32, LANES)
    idxs = [jnp.bitwise_and(iota + k, LANES - 1) for k in (8, 4, 2, 1)]
    bounds = [jnp.int32(0)]
    for s in range(1, BATCH):
        a = cnt[s - 1]
        for idx in idxs:
            a = a + a.at[idx].get(mode="promise_in_bounds")
        bounds.append(a[0])
    bounds.append(jnp.int32(ROWS))

    def compute_chunk(k, buf):
        for s in range(BATCH):
            lo = jnp.clip(bounds[s] - k * CHUNK, 0, CHUNK)
            hi = jnp.clip(bounds[s + 1] - k * CHUNK, 0, CHUNK)

            @pl.when(hi > lo)
            def _run(s=s, lo=lo, hi=hi, buf=buf):
                init = tuple(
                    acc_v[s, pl.ds(c * LANES, LANES)] for c in range(CVEC))

                @plsc.parallel_loop(lo, hi, 1, unroll=4, carry=init)
                def res(r, carry):
                    return tuple(
                        jnp.maximum(carry[c], buf[r, pl.ds(c * LANES, LANES)])
                        for c in range(CVEC))
                for c in range(CVEC):
                    acc_v[s, pl.ds(c * LANES, LANES)] = res[c]

    def gbody(g, _):
        k0 = 2 * g
        pltpu.make_async_copy(
            patches_hbm.at[pl.ds(base + k0 * CHUNK, CHUNK)], buf0,
            sem0).wait()
        compute_chunk(k0, buf0)

        @pl.when(k0 + 2 < NCHUNK)
        def _():
            pltpu.async_copy(
                patches_hbm.at[pl.ds(base + (k0 + 2) * CHUNK, CHUNK)],
                buf0, sem0)

        k1 = 2 * g + 1
        pltpu.make_async_copy(
            patches_hbm.at[pl.ds(base + k1 * CHUNK, CHUNK)], buf1,
            sem1).wait()
        compute_chunk(k1, buf1)

        @pl.when(k1 + 2 < NCHUNK)
        def _():
            pltpu.async_copy(
                patches_hbm.at[pl.ds(base + (k1 + 2) * CHUNK, CHUNK)],
                buf1, sem1)
        return 0
    lax.fori_loop(0, NCHUNK // 2, gbody, 0)

    pltpu.sync_copy(acc_v, parts_hbm.at[wid])


_sc_partials = pl.kernel(
    _sc_body,
    out_type=jax.ShapeDtypeStruct((NW, BATCH, C), jnp.float32),
    mesh=plsc.VectorSubcoreMesh(core_axis_name="c", subcore_axis_name="s",
                                num_cores=NUM_CORES,
                                num_subcores=NUM_SUBCORES),
    scratch_types=[
        pltpu.VMEM((ROWS,), jnp.int32),
        pltpu.VMEM((CHUNK, C), jnp.float32),
        pltpu.VMEM((CHUNK, C), jnp.float32),
        pltpu.VMEM((BATCH, C), jnp.float32),
        pltpu.SemaphoreType.DMA,
        pltpu.SemaphoreType.DMA,
    ],
)


def _tc_merge_body(parts_ref, out_ref):
    out_ref[:] = jnp.max(parts_ref[:], axis=0)


def kernel(patches, patch_ids):
    parts = _sc_partials(patches, patch_ids)
    out = pl.pallas_call(
        _tc_merge_body,
        out_shape=jax.ShapeDtypeStruct((BATCH, C), jnp.float32),
    )(parts)
    return out
